# Initial kernel scaffold; baseline (speedup 1.0000x reference)
import jax
import jax.numpy as jnp
from jax.experimental import pallas as pl


def kernel(x_gene, x_cell_type, edge_index_marker, edge_index_rev, edge_index_gene_self, edge_index_cell_self, params, lin):
    raise NotImplementedError("write your pallas kernel here")


# speedup vs baseline: 7.4399x; 7.4399x over previous
import functools

import jax
import jax.numpy as jnp
from jax import lax
from jax.experimental import pallas as pl
from jax.experimental.pallas import tpu as pltpu, tpu_sc as plsc

N_NODE = 10000
NP = 10112
D = 128
EP = 344064
OUT = 64

_MM = functools.partial(jnp.dot, preferred_element_type=jnp.float32,
                        precision=lax.Precision.HIGHEST)


def _stage_a_body(xg_ref, xc_ref, w_ref, b_ref, y_ref, r_ref):
    xg = xg_ref[...]
    xc = xc_ref[...]
    y_ref[0] = _MM(xg, w_ref[0])
    y_ref[1] = _MM(xc, w_ref[1])
    y_ref[2] = _MM(xc, w_ref[2])
    y_ref[3] = _MM(xg, w_ref[3])
    r_ref[0] = _MM(xc, w_ref[4]) + b_ref[0]
    r_ref[1] = _MM(xg, w_ref[5]) + b_ref[1]


def _stage_b_body(a_ref, w_ref, b_ref, y_ref, r_ref):
    hc = jnp.maximum(a_ref[0], 0.0)
    hg = jnp.maximum(a_ref[1], 0.0)
    y_ref[0] = _MM(hg, w_ref[0])
    y_ref[1] = _MM(hc, w_ref[1])
    r_ref[...] = _MM(hc, w_ref[2]) + b_ref[0]


def _stage_c_body(a_ref, w_ref, b_ref, o_ref):
    h = jnp.maximum(a_ref[0] + a_ref[1], 0.0)
    o_ref[...] = _MM(h, w_ref[...]) + b_ref[0]


_ROWS_BLK = 1000
_GRID = N_NODE // _ROWS_BLK


def _stage_a(xg, xc, w6, b2):
    return pl.pallas_call(
        _stage_a_body,
        grid=(_GRID,),
        in_specs=[
            pl.BlockSpec((_ROWS_BLK, D), lambda i: (i, 0)),
            pl.BlockSpec((_ROWS_BLK, D), lambda i: (i, 0)),
            pl.BlockSpec((6, D, D), lambda i: (0, 0, 0)),
            pl.BlockSpec((2, D), lambda i: (0, 0)),
        ],
        out_specs=[
            pl.BlockSpec((4, _ROWS_BLK, D), lambda i: (0, i, 0)),
            pl.BlockSpec((2, _ROWS_BLK, D), lambda i: (0, i, 0)),
        ],
        out_shape=[
            jax.ShapeDtypeStruct((4, N_NODE, D), jnp.float32),
            jax.ShapeDtypeStruct((2, N_NODE, D), jnp.float32),
        ],
    )(xg, xc, w6, b2)


def _stage_b(acc, w3, b1):
    return pl.pallas_call(
        _stage_b_body,
        grid=(_GRID,),
        in_specs=[
            pl.BlockSpec((2, _ROWS_BLK, D), lambda i: (0, i, 0)),
            pl.BlockSpec((3, D, D), lambda i: (0, 0, 0)),
            pl.BlockSpec((1, D), lambda i: (0, 0)),
        ],
        out_specs=[
            pl.BlockSpec((2, _ROWS_BLK, D), lambda i: (0, i, 0)),
            pl.BlockSpec((_ROWS_BLK, D), lambda i: (i, 0)),
        ],
        out_shape=[
            jax.ShapeDtypeStruct((2, N_NODE, D), jnp.float32),
            jax.ShapeDtypeStruct((N_NODE, D), jnp.float32),
        ],
    )(acc, w3, b1)


def _stage_c(acc, w, b1):
    return pl.pallas_call(
        _stage_c_body,
        grid=(_GRID,),
        in_specs=[
            pl.BlockSpec((2, _ROWS_BLK, D), lambda i: (0, i, 0)),
            pl.BlockSpec((D, D), lambda i: (0, 0)),
            pl.BlockSpec((1, D), lambda i: (0, 0)),
        ],
        out_specs=pl.BlockSpec((_ROWS_BLK, D), lambda i: (i, 0)),
        out_shape=jax.ShapeDtypeStruct((N_NODE, D), jnp.float32),
    )(acc, w, b1)


def _make_sc_segsum(kc, kb, core0_only=False):
    rows_per_tile = NP // 16
    nb = kc // kb
    mesh = plsc.VectorSubcoreMesh(core_axis_name="c", subcore_axis_name="s")

    @functools.partial(
        pl.kernel,
        mesh=mesh,
        out_type=jax.ShapeDtypeStruct((2, NP, D), jnp.float32),
        scratch_types=[
            pltpu.VMEM((kb, 128), jnp.int32),
            pltpu.VMEM((kb, 128), jnp.int32),
            pltpu.VMEM((128, D), jnp.float32),
            pltpu.VMEM((128, D), jnp.float32),
            pltpu.VMEM_SHARED((NP, D), jnp.float32),
            pltpu.SemaphoreType.DMA,
            pltpu.SemaphoreType.DMA,
        ],
    )
    def seg(src_hbm, dst_hbm, table_hbm, init_hbm, out_hbm,
            src_v, dst_v, rows_a, rows_b, acc_sh, sem_a, sem_b):
        c = lax.axis_index("c")
        s = lax.axis_index("s")
        row0 = s * rows_per_tile
        pltpu.sync_copy(init_hbm.at[c, pl.ds(row0, rows_per_tile)],
                        acc_sh.at[pl.ds(row0, rows_per_tile)])
        plsc.subcore_barrier()

        def run_edges():
            for b in range(nb):
                pltpu.sync_copy(src_hbm.at[c, s, pl.ds(b * kb, kb)], src_v)
                pltpu.sync_copy(dst_hbm.at[c, s, pl.ds(b * kb, kb)], dst_v)

                pltpu.async_copy(table_hbm.at[src_v.at[0]], rows_a, sem_a)
                pltpu.async_copy(table_hbm.at[src_v.at[1]], rows_b, sem_b)

                def body(i, carry):
                    j = i * 2
                    pltpu.make_async_copy(table_hbm.at[src_v.at[j]], rows_a,
                                          sem_a).wait()
                    pltpu.sync_copy(rows_a, acc_sh.at[dst_v.at[j]], add=True)
                    pltpu.async_copy(table_hbm.at[src_v.at[j + 2]], rows_a,
                                     sem_a)
                    pltpu.make_async_copy(table_hbm.at[src_v.at[j + 1]],
                                          rows_b, sem_b).wait()
                    pltpu.sync_copy(rows_b, acc_sh.at[dst_v.at[j + 1]],
                                    add=True)
                    pltpu.async_copy(table_hbm.at[src_v.at[j + 3]], rows_b,
                                     sem_b)
                    return carry

                lax.fori_loop(0, (kb - 2) // 2, body, 0)

                pltpu.make_async_copy(table_hbm.at[src_v.at[kb - 2]], rows_a,
                                      sem_a).wait()
                pltpu.sync_copy(rows_a, acc_sh.at[dst_v.at[kb - 2]], add=True)
                pltpu.make_async_copy(table_hbm.at[src_v.at[kb - 1]], rows_b,
                                      sem_b).wait()
                pltpu.sync_copy(rows_b, acc_sh.at[dst_v.at[kb - 1]], add=True)

        if core0_only:
            pl.when(c == 0)(run_edges)
        else:
            run_edges()

        plsc.subcore_barrier()
        pltpu.sync_copy(acc_sh.at[pl.ds(row0, rows_per_tile)],
                        out_hbm.at[c, pl.ds(row0, rows_per_tile)])

    return seg


_SC_L1 = _make_sc_segsum(EP // (16 * 128), 56)
_SC_L2 = _make_sc_segsum(EP // (16 * 128), 56, core0_only=True)


def _pad_edges(src, dst):
    pad_n = EP - src.shape[0]
    ar = jnp.arange(pad_n, dtype=jnp.int32)
    src = jnp.concatenate([src, ar % 8192])
    dst = jnp.concatenate([dst, N_NODE + (ar % 16)])
    return src, dst


def kernel(x_gene, x_cell_type, edge_index_marker, edge_index_rev,
           edge_index_gene_self, edge_index_cell_self, params, lin):
    src_c = jnp.concatenate([edge_index_marker[0],
                             edge_index_cell_self[0] + N_NODE])
    dst_c = jnp.concatenate([edge_index_marker[1], edge_index_cell_self[1]])
    src_g = jnp.concatenate([edge_index_rev[0] + 2 * N_NODE,
                             edge_index_gene_self[0] + 3 * N_NODE])
    dst_g = jnp.concatenate([edge_index_rev[1], edge_index_gene_self[1]])
    src_c, dst_c = _pad_edges(src_c, dst_c)
    src_g, dst_g = _pad_edges(src_g, dst_g)

    kc1 = EP // (16 * 128)
    src1 = jnp.stack([src_c, src_g]).reshape(2, 16, kc1, 128)
    dst1 = jnp.stack([dst_c, dst_g]).reshape(2, 16, kc1, 128)

    p0, p1 = params
    w6 = jnp.stack([
        p0["marker"]["W_rel"].T, p0["cell_self"]["W_rel"].T,
        p0["rev"]["W_rel"].T, p0["gene_self"]["W_rel"].T,
        (p0["marker"]["W_root"] + p0["cell_self"]["W_root"]).T,
        (p0["rev"]["W_root"] + p0["gene_self"]["W_root"]).T,
    ])
    b2 = jnp.stack([
        p0["marker"]["b_rel"] + p0["cell_self"]["b_rel"],
        p0["rev"]["b_rel"] + p0["gene_self"]["b_rel"],
    ])
    w3 = jnp.stack([
        p1["marker"]["W_rel"].T, p1["cell_self"]["W_rel"].T,
        (p1["marker"]["W_root"] + p1["cell_self"]["W_root"]).T,
    ])
    b1 = (p1["marker"]["b_rel"] + p1["cell_self"]["b_rel"]).reshape(1, D)
    w_lin = jnp.zeros((D, D), jnp.float32).at[:, :OUT].set(lin["W"].T)
    b_lin = jnp.zeros((1, D), jnp.float32).at[0, :OUT].set(lin["b"])

    y0, r0 = _stage_a(x_gene, x_cell_type, w6, b2)
    r0p = jnp.pad(r0, ((0, 0), (0, NP - N_NODE), (0, 0)))
    acc1 = _SC_L1(src1, dst1, y0.reshape(4 * N_NODE, D), r0p)

    y1, r1 = _stage_b(acc1[:, :N_NODE], w3, b1)
    init2 = jnp.concatenate([
        jnp.pad(r1, ((0, NP - N_NODE), (0, 0)))[None],
        jnp.zeros((1, NP, D), jnp.float32),
    ])
    acc2 = _SC_L2(src1, dst1, y1.reshape(2 * N_NODE, D), init2)

    out = _stage_c(acc2[:, :N_NODE], w_lin, b_lin)
    return out[:, :OUT]


# speedup vs baseline: 8.6581x; 1.1637x over previous
import functools

import jax
import jax.numpy as jnp
from jax import lax
from jax.experimental import pallas as pl
from jax.experimental.pallas import tpu as pltpu, tpu_sc as plsc

N_NODE = 10000
NP = 10112
D = 128
EP = 344064
OUT = 64

_MM = functools.partial(jnp.dot, preferred_element_type=jnp.float32,
                        precision=lax.Precision.HIGHEST)


def _stage_a_body(xg_ref, xc_ref, w_ref, b_ref, y_ref, r_ref):
    xg = xg_ref[...]
    xc = xc_ref[...]
    y_ref[0] = _MM(xg, w_ref[0])
    y_ref[1] = _MM(xc, w_ref[1])
    y_ref[2] = _MM(xc, w_ref[2])
    y_ref[3] = _MM(xg, w_ref[3])
    r_ref[0] = _MM(xc, w_ref[4]) + b_ref[0]
    r_ref[1] = _MM(xg, w_ref[5]) + b_ref[1]


def _stage_b_body(a_ref, w_ref, b_ref, y_ref, r_ref):
    hc = jnp.maximum(a_ref[0], 0.0)
    hg = jnp.maximum(a_ref[1], 0.0)
    y_ref[0] = _MM(hg, w_ref[0])
    y_ref[1] = _MM(hc, w_ref[1])
    r_ref[0] = _MM(hc, w_ref[2]) + b_ref[0]
    r_ref[1] = jnp.zeros((_ROWS_BLK, D), jnp.float32)


def _stage_c_body(a_ref, w_ref, b_ref, o_ref):
    h = jnp.maximum(a_ref[0] + a_ref[1], 0.0)
    o_ref[...] = _MM(h, w_ref[...]) + b_ref[0]


_ROWS_BLK = 1000
_GRID = N_NODE // _ROWS_BLK


def _stage_a(xg, xc, w6, b2):
    return pl.pallas_call(
        _stage_a_body,
        grid=(_GRID,),
        in_specs=[
            pl.BlockSpec((_ROWS_BLK, D), lambda i: (i, 0)),
            pl.BlockSpec((_ROWS_BLK, D), lambda i: (i, 0)),
            pl.BlockSpec((6, D, D), lambda i: (0, 0, 0)),
            pl.BlockSpec((2, D), lambda i: (0, 0)),
        ],
        out_specs=[
            pl.BlockSpec((4, _ROWS_BLK, D), lambda i: (0, i, 0)),
            pl.BlockSpec((2, _ROWS_BLK, D), lambda i: (0, i, 0)),
        ],
        out_shape=[
            jax.ShapeDtypeStruct((4, N_NODE, D), jnp.float32),
            jax.ShapeDtypeStruct((2, NP, D), jnp.float32),
        ],
    )(xg, xc, w6, b2)


def _stage_b(acc, w3, b1):
    return pl.pallas_call(
        _stage_b_body,
        grid=(_GRID,),
        in_specs=[
            pl.BlockSpec((2, _ROWS_BLK, D), lambda i: (0, i, 0)),
            pl.BlockSpec((3, D, D), lambda i: (0, 0, 0)),
            pl.BlockSpec((1, D), lambda i: (0, 0)),
        ],
        out_specs=[
            pl.BlockSpec((2, _ROWS_BLK, D), lambda i: (0, i, 0)),
            pl.BlockSpec((2, _ROWS_BLK, D), lambda i: (0, i, 0)),
        ],
        out_shape=[
            jax.ShapeDtypeStruct((2, N_NODE, D), jnp.float32),
            jax.ShapeDtypeStruct((2, NP, D), jnp.float32),
        ],
    )(acc, w3, b1)


def _stage_c(acc, w, b1):
    return pl.pallas_call(
        _stage_c_body,
        grid=(_GRID,),
        in_specs=[
            pl.BlockSpec((2, _ROWS_BLK, D), lambda i: (0, i, 0)),
            pl.BlockSpec((D, D), lambda i: (0, 0)),
            pl.BlockSpec((1, D), lambda i: (0, 0)),
        ],
        out_specs=pl.BlockSpec((_ROWS_BLK, D), lambda i: (i, 0)),
        out_shape=jax.ShapeDtypeStruct((N_NODE, D), jnp.float32),
    )(acc, w, b1)


def _make_sc_segsum(kc, kb, chunk):
    rows_per_tile = NP // 16
    nb = kc // kb
    mesh = plsc.VectorSubcoreMesh(core_axis_name="c", subcore_axis_name="s")

    @functools.partial(
        pl.kernel,
        mesh=mesh,
        out_type=jax.ShapeDtypeStruct((2, NP, D), jnp.float32),
        scratch_types=[
            pltpu.VMEM((kb, chunk), jnp.int32),
            pltpu.VMEM((kb, chunk), jnp.int32),
            pltpu.VMEM((chunk, D), jnp.float32),
            pltpu.VMEM((chunk, D), jnp.float32),
            pltpu.VMEM_SHARED((NP, D), jnp.float32),
            pltpu.SemaphoreType.DMA,
            pltpu.SemaphoreType.DMA,
        ],
    )
    def seg(src_hbm, dst_hbm, table_hbm, init_hbm, out_hbm,
            src_v, dst_v, rows_a, rows_b, acc_sh, sem_a, sem_b):
        c = lax.axis_index("c")
        s = lax.axis_index("s")
        row0 = s * rows_per_tile
        pltpu.sync_copy(init_hbm.at[c, pl.ds(row0, rows_per_tile)],
                        acc_sh.at[pl.ds(row0, rows_per_tile)])
        plsc.subcore_barrier()

        def run_edges():
            for b in range(nb):
                pltpu.sync_copy(src_hbm.at[c, s, pl.ds(b * kb, kb)], src_v)
                pltpu.sync_copy(dst_hbm.at[c, s, pl.ds(b * kb, kb)], dst_v)

                pltpu.async_copy(table_hbm.at[src_v.at[0]], rows_a, sem_a)
                pltpu.async_copy(table_hbm.at[src_v.at[1]], rows_b, sem_b)

                def body(i, carry):
                    j = i * 2
                    pltpu.make_async_copy(table_hbm.at[src_v.at[j]], rows_a,
                                          sem_a).wait()
                    pltpu.sync_copy(rows_a, acc_sh.at[dst_v.at[j]], add=True)
                    pltpu.async_copy(table_hbm.at[src_v.at[j + 2]], rows_a,
                                     sem_a)
                    pltpu.make_async_copy(table_hbm.at[src_v.at[j + 1]],
                                          rows_b, sem_b).wait()
                    pltpu.sync_copy(rows_b, acc_sh.at[dst_v.at[j + 1]],
                                    add=True)
                    pltpu.async_copy(table_hbm.at[src_v.at[j + 3]], rows_b,
                                     sem_b)
                    return carry

                lax.fori_loop(0, (kb - 2) // 2, body, 0)

                pltpu.make_async_copy(table_hbm.at[src_v.at[kb - 2]], rows_a,
                                      sem_a).wait()
                pltpu.sync_copy(rows_a, acc_sh.at[dst_v.at[kb - 2]], add=True)
                pltpu.make_async_copy(table_hbm.at[src_v.at[kb - 1]], rows_b,
                                      sem_b).wait()
                pltpu.sync_copy(rows_b, acc_sh.at[dst_v.at[kb - 1]], add=True)

        run_edges()

        plsc.subcore_barrier()
        pltpu.sync_copy(acc_sh.at[pl.ds(row0, rows_per_tile)],
                        out_hbm.at[c, pl.ds(row0, rows_per_tile)])

    return seg


_SC_L1 = _make_sc_segsum(EP // (16 * 128), 56, 128)
_SC_L2 = _make_sc_segsum(EP // (32 * 64), 56, 64)


def _pad_edges(src, dst):
    pad_n = EP - src.shape[0]
    ar = jnp.arange(pad_n, dtype=jnp.int32)
    src = jnp.concatenate([src, ar % 8192])
    dst = jnp.concatenate([dst, N_NODE + (ar % 16)])
    return src, dst


def kernel(x_gene, x_cell_type, edge_index_marker, edge_index_rev,
           edge_index_gene_self, edge_index_cell_self, params, lin):
    src_c = jnp.concatenate([edge_index_marker[0],
                             edge_index_cell_self[0] + N_NODE])
    dst_c = jnp.concatenate([edge_index_marker[1], edge_index_cell_self[1]])
    src_g = jnp.concatenate([edge_index_rev[0] + 2 * N_NODE,
                             edge_index_gene_self[0] + 3 * N_NODE])
    dst_g = jnp.concatenate([edge_index_rev[1], edge_index_gene_self[1]])
    src_c, dst_c = _pad_edges(src_c, dst_c)
    src_g, dst_g = _pad_edges(src_g, dst_g)

    kc1 = EP // (16 * 128)
    src1 = jnp.stack([src_c, src_g]).reshape(2, 16, kc1, 128)
    dst1 = jnp.stack([dst_c, dst_g]).reshape(2, 16, kc1, 128)
    kc2 = EP // (32 * 64)
    src2 = src_c.reshape(2, 16, kc2, 64)
    dst2 = dst_c.reshape(2, 16, kc2, 64)

    p0, p1 = params
    w6 = jnp.stack([
        p0["marker"]["W_rel"].T, p0["cell_self"]["W_rel"].T,
        p0["rev"]["W_rel"].T, p0["gene_self"]["W_rel"].T,
        (p0["marker"]["W_root"] + p0["cell_self"]["W_root"]).T,
        (p0["rev"]["W_root"] + p0["gene_self"]["W_root"]).T,
    ])
    b2 = jnp.stack([
        p0["marker"]["b_rel"] + p0["cell_self"]["b_rel"],
        p0["rev"]["b_rel"] + p0["gene_self"]["b_rel"],
    ])
    w3 = jnp.stack([
        p1["marker"]["W_rel"].T, p1["cell_self"]["W_rel"].T,
        (p1["marker"]["W_root"] + p1["cell_self"]["W_root"]).T,
    ])
    b1 = (p1["marker"]["b_rel"] + p1["cell_self"]["b_rel"]).reshape(1, D)
    w_lin = jnp.zeros((D, D), jnp.float32).at[:, :OUT].set(lin["W"].T)
    b_lin = jnp.zeros((1, D), jnp.float32).at[0, :OUT].set(lin["b"])

    y0, r0 = _stage_a(x_gene, x_cell_type, w6, b2)
    acc1 = _SC_L1(src1, dst1, y0.reshape(4 * N_NODE, D), r0)

    y1, r1 = _stage_b(acc1, w3, b1)
    acc2 = _SC_L2(src2, dst2, y1.reshape(2 * N_NODE, D), r1)

    out = _stage_c(acc2, w_lin, b_lin)
    return out[:, :OUT]


# speedup vs baseline: 9.6696x; 1.1168x over previous
import functools

import jax
import jax.numpy as jnp
from jax import lax
from jax.experimental import pallas as pl
from jax.experimental.pallas import tpu as pltpu, tpu_sc as plsc

N_NODE = 10000
NP = 10112
D = 128
EP = 344064
OUT = 64

_MM = functools.partial(jnp.dot, preferred_element_type=jnp.float32,
                        precision=lax.Precision.DEFAULT)


def _stage_a_body(xg_ref, xc_ref, wg_ref, wc_ref, b_ref, y_ref, r_ref):
    og = _MM(xg_ref[...], wg_ref[...])
    oc = _MM(xc_ref[...], wc_ref[...])
    y_ref[0] = og[:, :D]
    y_ref[1] = oc[:, :D]
    y_ref[2] = oc[:, D:2 * D]
    y_ref[3] = og[:, D:2 * D]
    r_ref[0] = oc[:, 2 * D:] + b_ref[0]
    r_ref[1] = og[:, 2 * D:] + b_ref[1]


def _stage_b_body(a_ref, wg_ref, wc_ref, b_ref, y_ref, r_ref):
    hc = jnp.maximum(a_ref[0], 0.0)
    hg = jnp.maximum(a_ref[1], 0.0)
    oc = _MM(hc, wc_ref[...])
    y_ref[0] = _MM(hg, wg_ref[...])
    y_ref[1] = oc[:, :D]
    r_ref[0] = oc[:, D:] + b_ref[0]
    r_ref[1] = jnp.zeros((_ROWS_BLK, D), jnp.float32)


def _stage_c_body(a_ref, w_ref, b_ref, o_ref):
    h = jnp.maximum(a_ref[0] + a_ref[1], 0.0)
    o_ref[...] = _MM(h, w_ref[...]) + b_ref[0]


_ROWS_BLK = 1000
_GRID = N_NODE // _ROWS_BLK


def _stage_a(xg, xc, wg, wc, b2):
    return pl.pallas_call(
        _stage_a_body,
        grid=(_GRID,),
        in_specs=[
            pl.BlockSpec((_ROWS_BLK, D), lambda i: (i, 0)),
            pl.BlockSpec((_ROWS_BLK, D), lambda i: (i, 0)),
            pl.BlockSpec((D, 3 * D), lambda i: (0, 0)),
            pl.BlockSpec((D, 3 * D), lambda i: (0, 0)),
            pl.BlockSpec((2, D), lambda i: (0, 0)),
        ],
        out_specs=[
            pl.BlockSpec((4, _ROWS_BLK, D), lambda i: (0, i, 0)),
            pl.BlockSpec((2, _ROWS_BLK, D), lambda i: (0, i, 0)),
        ],
        out_shape=[
            jax.ShapeDtypeStruct((4, N_NODE, D), jnp.float32),
            jax.ShapeDtypeStruct((2, NP, D), jnp.float32),
        ],
    )(xg, xc, wg, wc, b2)


def _stage_b(acc, wg, wc, b1):
    return pl.pallas_call(
        _stage_b_body,
        grid=(_GRID,),
        in_specs=[
            pl.BlockSpec((2, _ROWS_BLK, D), lambda i: (0, i, 0)),
            pl.BlockSpec((D, D), lambda i: (0, 0)),
            pl.BlockSpec((D, 2 * D), lambda i: (0, 0)),
            pl.BlockSpec((1, D), lambda i: (0, 0)),
        ],
        out_specs=[
            pl.BlockSpec((2, _ROWS_BLK, D), lambda i: (0, i, 0)),
            pl.BlockSpec((2, _ROWS_BLK, D), lambda i: (0, i, 0)),
        ],
        out_shape=[
            jax.ShapeDtypeStruct((2, N_NODE, D), jnp.float32),
            jax.ShapeDtypeStruct((2, NP, D), jnp.float32),
        ],
    )(acc, wg, wc, b1)


def _stage_c(acc, w, b1):
    return pl.pallas_call(
        _stage_c_body,
        grid=(_GRID,),
        in_specs=[
            pl.BlockSpec((2, _ROWS_BLK, D), lambda i: (0, i, 0)),
            pl.BlockSpec((D, D), lambda i: (0, 0)),
            pl.BlockSpec((1, D), lambda i: (0, 0)),
        ],
        out_specs=pl.BlockSpec((_ROWS_BLK, D), lambda i: (i, 0)),
        out_shape=jax.ShapeDtypeStruct((N_NODE, D), jnp.float32),
    )(acc, w, b1)


def _make_sc_segsum(kc, kb, chunk):
    rows_per_tile = NP // 16
    nb = kc // kb
    mesh = plsc.VectorSubcoreMesh(core_axis_name="c", subcore_axis_name="s")

    @functools.partial(
        pl.kernel,
        mesh=mesh,
        out_type=jax.ShapeDtypeStruct((2, NP, D), jnp.float32),
        scratch_types=[
            pltpu.VMEM((kb, chunk), jnp.int32),
            pltpu.VMEM((kb, chunk), jnp.int32),
            pltpu.VMEM((chunk, D), jnp.float32),
            pltpu.VMEM((chunk, D), jnp.float32),
            pltpu.VMEM_SHARED((NP, D), jnp.float32),
            pltpu.SemaphoreType.DMA,
            pltpu.SemaphoreType.DMA,
        ],
    )
    def seg(src_hbm, dst_hbm, table_hbm, init_hbm, out_hbm,
            src_v, dst_v, rows_a, rows_b, acc_sh, sem_a, sem_b):
        c = lax.axis_index("c")
        s = lax.axis_index("s")
        row0 = s * rows_per_tile
        pltpu.sync_copy(init_hbm.at[c, pl.ds(row0, rows_per_tile)],
                        acc_sh.at[pl.ds(row0, rows_per_tile)])
        plsc.subcore_barrier()

        def run_edges():
            for b in range(nb):
                pltpu.sync_copy(src_hbm.at[c, s, pl.ds(b * kb, kb)], src_v)
                pltpu.sync_copy(dst_hbm.at[c, s, pl.ds(b * kb, kb)], dst_v)

                pltpu.async_copy(table_hbm.at[src_v.at[0]], rows_a, sem_a)
                pltpu.async_copy(table_hbm.at[src_v.at[1]], rows_b, sem_b)

                def body(i, carry):
                    j = i * 2
                    pltpu.make_async_copy(table_hbm.at[src_v.at[j]], rows_a,
                                          sem_a).wait()
                    pltpu.sync_copy(rows_a, acc_sh.at[dst_v.at[j]], add=True)
                    pltpu.async_copy(table_hbm.at[src_v.at[j + 2]], rows_a,
                                     sem_a)
                    pltpu.make_async_copy(table_hbm.at[src_v.at[j + 1]],
                                          rows_b, sem_b).wait()
                    pltpu.sync_copy(rows_b, acc_sh.at[dst_v.at[j + 1]],
                                    add=True)
                    pltpu.async_copy(table_hbm.at[src_v.at[j + 3]], rows_b,
                                     sem_b)
                    return carry

                lax.fori_loop(0, (kb - 2) // 2, body, 0)

                pltpu.make_async_copy(table_hbm.at[src_v.at[kb - 2]], rows_a,
                                      sem_a).wait()
                pltpu.sync_copy(rows_a, acc_sh.at[dst_v.at[kb - 2]], add=True)
                pltpu.make_async_copy(table_hbm.at[src_v.at[kb - 1]], rows_b,
                                      sem_b).wait()
                pltpu.sync_copy(rows_b, acc_sh.at[dst_v.at[kb - 1]], add=True)

        run_edges()

        plsc.subcore_barrier()
        pltpu.sync_copy(acc_sh.at[pl.ds(row0, rows_per_tile)],
                        out_hbm.at[c, pl.ds(row0, rows_per_tile)])

    return seg


_SC_L1 = _make_sc_segsum(EP // (16 * 128), 56, 128)
_SC_L2 = _make_sc_segsum(EP // (32 * 64), 56, 64)


def _pad_edges(src, dst):
    pad_n = EP - src.shape[0]
    ar = jnp.arange(pad_n, dtype=jnp.int32)
    src = jnp.concatenate([src, ar % 8192])
    dst = jnp.concatenate([dst, N_NODE + (ar % 16)])
    return src, dst


def kernel(x_gene, x_cell_type, edge_index_marker, edge_index_rev,
           edge_index_gene_self, edge_index_cell_self, params, lin):
    src_c = jnp.concatenate([edge_index_marker[0],
                             edge_index_cell_self[0] + N_NODE])
    dst_c = jnp.concatenate([edge_index_marker[1], edge_index_cell_self[1]])
    src_g = jnp.concatenate([edge_index_rev[0] + 2 * N_NODE,
                             edge_index_gene_self[0] + 3 * N_NODE])
    dst_g = jnp.concatenate([edge_index_rev[1], edge_index_gene_self[1]])
    src_c, dst_c = _pad_edges(src_c, dst_c)
    src_g, dst_g = _pad_edges(src_g, dst_g)

    kc1 = EP // (16 * 128)
    src1 = jnp.stack([src_c, src_g]).reshape(2, 16, kc1, 128)
    dst1 = jnp.stack([dst_c, dst_g]).reshape(2, 16, kc1, 128)
    kc2 = EP // (32 * 64)
    src2 = src_c.reshape(2, 16, kc2, 64)
    dst2 = dst_c.reshape(2, 16, kc2, 64)

    p0, p1 = params
    wg0 = jnp.concatenate([
        p0["marker"]["W_rel"].T, p0["gene_self"]["W_rel"].T,
        (p0["rev"]["W_root"] + p0["gene_self"]["W_root"]).T,
    ], axis=1)
    wc0 = jnp.concatenate([
        p0["cell_self"]["W_rel"].T, p0["rev"]["W_rel"].T,
        (p0["marker"]["W_root"] + p0["cell_self"]["W_root"]).T,
    ], axis=1)
    b2 = jnp.stack([
        p0["marker"]["b_rel"] + p0["cell_self"]["b_rel"],
        p0["rev"]["b_rel"] + p0["gene_self"]["b_rel"],
    ])
    wg1 = p1["marker"]["W_rel"].T
    wc1 = jnp.concatenate([
        p1["cell_self"]["W_rel"].T,
        (p1["marker"]["W_root"] + p1["cell_self"]["W_root"]).T,
    ], axis=1)
    b1 = (p1["marker"]["b_rel"] + p1["cell_self"]["b_rel"]).reshape(1, D)
    w_lin = jnp.zeros((D, D), jnp.float32).at[:, :OUT].set(lin["W"].T)
    b_lin = jnp.zeros((1, D), jnp.float32).at[0, :OUT].set(lin["b"])

    y0, r0 = _stage_a(x_gene, x_cell_type, wg0, wc0, b2)
    acc1 = _SC_L1(src1, dst1, y0.reshape(4 * N_NODE, D), r0)

    y1, r1 = _stage_b(acc1, wg1, wc1, b1)
    acc2 = _SC_L2(src2, dst2, y1.reshape(2 * N_NODE, D), r1)

    out = _stage_c(acc2, w_lin, b_lin)
    return out[:, :OUT]


# speedup vs baseline: 9.9872x; 1.0328x over previous
import functools

import jax
import jax.numpy as jnp
from jax import lax
from jax.experimental import pallas as pl
from jax.experimental.pallas import tpu as pltpu, tpu_sc as plsc

N_NODE = 10000
NP = 10112
D = 128
OUT = 64

E_BIP = 320000
E_SELF = 10000
CH_M = 2560
CH_S = 128
PAD_M = CH_M * 128 - E_BIP
PAD_S = CH_S * 128 - E_SELF
OFF_M, OFF_R, OFF_C, OFF_G = 0, CH_M, 2 * CH_M, 2 * CH_M + CH_S
CH_TOT = 2 * CH_M + 2 * CH_S
KB_MAX = 40

_MM = functools.partial(jnp.dot, preferred_element_type=jnp.float32,
                        precision=lax.Precision.DEFAULT)


def _stage_a_body(xg_ref, xc_ref, wg_ref, wc_ref, b_ref,
                  ym_ref, ycs_ref, yrev_ref, ygs_ref, r_ref):
    og = _MM(xg_ref[...], wg_ref[...])
    oc = _MM(xc_ref[...], wc_ref[...])
    ym_ref[...] = og[:, :D]
    ygs_ref[...] = og[:, D:2 * D]
    r_ref[1] = og[:, 2 * D:] + b_ref[1]
    ycs_ref[...] = oc[:, :D]
    yrev_ref[...] = oc[:, D:2 * D]
    r_ref[0] = oc[:, 2 * D:] + b_ref[0]


def _stage_b_body(a_ref, wg_ref, wc_ref, b_ref, ym_ref, ycs_ref, r_ref):
    hc = jnp.maximum(a_ref[0], 0.0)
    hg = jnp.maximum(a_ref[1], 0.0)
    oc = _MM(hc, wc_ref[...])
    ym_ref[...] = _MM(hg, wg_ref[...])
    ycs_ref[...] = oc[:, :D]
    r_ref[0] = oc[:, D:] + b_ref[0]
    r_ref[1] = jnp.zeros((_ROWS_BLK, D), jnp.float32)


def _stage_c_body(a_ref, w_ref, b_ref, o_ref):
    h = jnp.maximum(a_ref[0] + a_ref[1], 0.0)
    o_ref[...] = _MM(h, w_ref[...]) + b_ref[0]


_ROWS_BLK = 1000
_GRID = N_NODE // _ROWS_BLK
_TBL_SPEC = pl.BlockSpec((_ROWS_BLK, D), lambda i: (i, 0))
_TBL_SHAPE = jax.ShapeDtypeStruct((N_NODE, D), jnp.float32)
_R_SPEC = pl.BlockSpec((2, _ROWS_BLK, D), lambda i: (0, i, 0))
_R_SHAPE = jax.ShapeDtypeStruct((2, NP, D), jnp.float32)


def _stage_a(xg, xc, wg, wc, b2):
    return pl.pallas_call(
        _stage_a_body,
        grid=(_GRID,),
        in_specs=[
            pl.BlockSpec((_ROWS_BLK, D), lambda i: (i, 0)),
            pl.BlockSpec((_ROWS_BLK, D), lambda i: (i, 0)),
            pl.BlockSpec((D, 3 * D), lambda i: (0, 0)),
            pl.BlockSpec((D, 3 * D), lambda i: (0, 0)),
            pl.BlockSpec((2, D), lambda i: (0, 0)),
        ],
        out_specs=[_TBL_SPEC, _TBL_SPEC, _TBL_SPEC, _TBL_SPEC, _R_SPEC],
        out_shape=[_TBL_SHAPE, _TBL_SHAPE, _TBL_SHAPE, _TBL_SHAPE, _R_SHAPE],
    )(xg, xc, wg, wc, b2)


def _stage_b(acc, wg, wc, b1):
    return pl.pallas_call(
        _stage_b_body,
        grid=(_GRID,),
        in_specs=[
            pl.BlockSpec((2, _ROWS_BLK, D), lambda i: (0, i, 0)),
            pl.BlockSpec((D, D), lambda i: (0, 0)),
            pl.BlockSpec((D, 2 * D), lambda i: (0, 0)),
            pl.BlockSpec((1, D), lambda i: (0, 0)),
        ],
        out_specs=[_TBL_SPEC, _TBL_SPEC, _R_SPEC],
        out_shape=[_TBL_SHAPE, _TBL_SHAPE, _R_SHAPE],
    )(acc, wg, wc, b1)


def _stage_c(acc, w, b1):
    return pl.pallas_call(
        _stage_c_body,
        grid=(_GRID,),
        in_specs=[
            pl.BlockSpec((2, _ROWS_BLK, D), lambda i: (0, i, 0)),
            pl.BlockSpec((D, D), lambda i: (0, 0)),
            pl.BlockSpec((1, D), lambda i: (0, 0)),
        ],
        out_specs=pl.BlockSpec((_ROWS_BLK, D), lambda i: (i, 0)),
        out_shape=jax.ShapeDtypeStruct((N_NODE, D), jnp.float32),
    )(acc, w, b1)


def _run_rel(src2d, dst2d, tbl, src_v, dst_v, rows_a, rows_b,
             acc_sh, sem_a, sem_b, start, kb, nb):
    for b in range(nb):
        base = start + b * kb
        pltpu.sync_copy(src2d.at[pl.ds(base, kb)], src_v.at[pl.ds(0, kb)])
        pltpu.sync_copy(dst2d.at[pl.ds(base, kb)], dst_v.at[pl.ds(0, kb)])

        pltpu.async_copy(tbl.at[src_v.at[0]], rows_a, sem_a)
        pltpu.async_copy(tbl.at[src_v.at[1]], rows_b, sem_b)

        def body(i, carry):
            j = i * 2
            pltpu.make_async_copy(tbl.at[src_v.at[j]], rows_a, sem_a).wait()
            pltpu.sync_copy(rows_a, acc_sh.at[dst_v.at[j]], add=True)
            pltpu.async_copy(tbl.at[src_v.at[j + 2]], rows_a, sem_a)
            pltpu.make_async_copy(tbl.at[src_v.at[j + 1]], rows_b,
                                  sem_b).wait()
            pltpu.sync_copy(rows_b, acc_sh.at[dst_v.at[j + 1]], add=True)
            pltpu.async_copy(tbl.at[src_v.at[j + 3]], rows_b, sem_b)
            return carry

        lax.fori_loop(0, (kb - 2) // 2, body, 0)

        pltpu.make_async_copy(tbl.at[src_v.at[kb - 2]], rows_a, sem_a).wait()
        pltpu.sync_copy(rows_a, acc_sh.at[dst_v.at[kb - 2]], add=True)
        pltpu.make_async_copy(tbl.at[src_v.at[kb - 1]], rows_b, sem_b).wait()
        pltpu.sync_copy(rows_b, acc_sh.at[dst_v.at[kb - 1]], add=True)


_MESH = plsc.VectorSubcoreMesh(core_axis_name="c", subcore_axis_name="s")
_ROWS_PER_TILE = NP // 16
_SC_SCRATCH = [
    pltpu.VMEM((KB_MAX, 128), jnp.int32),
    pltpu.VMEM((KB_MAX, 128), jnp.int32),
    pltpu.VMEM((128, D), jnp.float32),
    pltpu.VMEM((128, D), jnp.float32),
    pltpu.VMEM_SHARED((NP, D), jnp.float32),
    pltpu.SemaphoreType.DMA,
    pltpu.SemaphoreType.DMA,
]


@functools.partial(
    pl.kernel, mesh=_MESH,
    out_type=jax.ShapeDtypeStruct((2, NP, D), jnp.float32),
    scratch_types=_SC_SCRATCH,
)
def _sc_layer1(src2d, dst2d, tm, tcs, trev, tgs, init_hbm, out_hbm,
               src_v, dst_v, rows_a, rows_b, acc_sh, sem_a, sem_b):
    c = lax.axis_index("c")
    s = lax.axis_index("s")
    row0 = s * _ROWS_PER_TILE
    pltpu.sync_copy(init_hbm.at[c, pl.ds(row0, _ROWS_PER_TILE)],
                    acc_sh.at[pl.ds(row0, _ROWS_PER_TILE)])
    plsc.subcore_barrier()

    args = (src_v, dst_v, rows_a, rows_b, acc_sh, sem_a, sem_b)

    @pl.when(c == 0)
    def _cell_side():
        _run_rel(src2d, dst2d, tm, *args, OFF_M + s * (CH_M // 16), 32, 5)
        _run_rel(src2d, dst2d, tcs, *args, OFF_C + s * (CH_S // 16), 8, 1)

    @pl.when(c == 1)
    def _gene_side():
        _run_rel(src2d, dst2d, trev, *args, OFF_R + s * (CH_M // 16), 32, 5)
        _run_rel(src2d, dst2d, tgs, *args, OFF_G + s * (CH_S // 16), 8, 1)

    plsc.subcore_barrier()
    pltpu.sync_copy(acc_sh.at[pl.ds(row0, _ROWS_PER_TILE)],
                    out_hbm.at[c, pl.ds(row0, _ROWS_PER_TILE)])


@functools.partial(
    pl.kernel, mesh=_MESH,
    out_type=jax.ShapeDtypeStruct((2, NP, D), jnp.float32),
    scratch_types=_SC_SCRATCH,
)
def _sc_layer2(src2d, dst2d, tm, tcs, init_hbm, out_hbm,
               src_v, dst_v, rows_a, rows_b, acc_sh, sem_a, sem_b):
    c = lax.axis_index("c")
    s = lax.axis_index("s")
    row0 = s * _ROWS_PER_TILE
    pltpu.sync_copy(init_hbm.at[c, pl.ds(row0, _ROWS_PER_TILE)],
                    acc_sh.at[pl.ds(row0, _ROWS_PER_TILE)])
    plsc.subcore_barrier()

    args = (src_v, dst_v, rows_a, rows_b, acc_sh, sem_a, sem_b)
    _run_rel(src2d, dst2d, tm, *args,
             OFF_M + c * (CH_M // 2) + s * (CH_M // 32), 40, 2)

    @pl.when(s < 8)
    def _cs():
        _run_rel(src2d, dst2d, tcs, *args, OFF_C + c * (CH_S // 2) + s * 8,
                 8, 1)

    plsc.subcore_barrier()
    pltpu.sync_copy(acc_sh.at[pl.ds(row0, _ROWS_PER_TILE)],
                    out_hbm.at[c, pl.ds(row0, _ROWS_PER_TILE)])


def kernel(x_gene, x_cell_type, edge_index_marker, edge_index_rev,
           edge_index_gene_self, edge_index_cell_self, params, lin):
    arm = jnp.arange(PAD_M, dtype=jnp.int32)
    ars = jnp.arange(PAD_S, dtype=jnp.int32)
    psrc_m, psrc_s = arm & 8191, ars & 8191
    pdst_m, pdst_s = N_NODE + (arm & 63), N_NODE + (ars & 63)
    srcs = jnp.concatenate([
        edge_index_marker[0], psrc_m, edge_index_rev[0], psrc_m,
        edge_index_cell_self[0], psrc_s, edge_index_gene_self[0], psrc_s,
    ]).reshape(CH_TOT, 128)
    dsts = jnp.concatenate([
        edge_index_marker[1], pdst_m, edge_index_rev[1], pdst_m,
        edge_index_cell_self[1], pdst_s, edge_index_gene_self[1], pdst_s,
    ]).reshape(CH_TOT, 128)

    p0, p1 = params
    wg0 = jnp.concatenate([
        p0["marker"]["W_rel"].T, p0["gene_self"]["W_rel"].T,
        (p0["rev"]["W_root"] + p0["gene_self"]["W_root"]).T,
    ], axis=1)
    wc0 = jnp.concatenate([
        p0["cell_self"]["W_rel"].T, p0["rev"]["W_rel"].T,
        (p0["marker"]["W_root"] + p0["cell_self"]["W_root"]).T,
    ], axis=1)
    b2 = jnp.stack([
        p0["marker"]["b_rel"] + p0["cell_self"]["b_rel"],
        p0["rev"]["b_rel"] + p0["gene_self"]["b_rel"],
    ])
    wg1 = p1["marker"]["W_rel"].T
    wc1 = jnp.concatenate([
        p1["cell_self"]["W_rel"].T,
        (p1["marker"]["W_root"] + p1["cell_self"]["W_root"]).T,
    ], axis=1)
    b1 = (p1["marker"]["b_rel"] + p1["cell_self"]["b_rel"]).reshape(1, D)
    w_lin = jnp.zeros((D, D), jnp.float32).at[:, :OUT].set(lin["W"].T)
    b_lin = jnp.zeros((1, D), jnp.float32).at[0, :OUT].set(lin["b"])

    ym0, ycs0, yrev0, ygs0, r0 = _stage_a(x_gene, x_cell_type, wg0, wc0, b2)
    acc1 = _sc_layer1(srcs, dsts, ym0, ycs0, yrev0, ygs0, r0)

    ym1, ycs1, r1 = _stage_b(acc1, wg1, wc1, b1)
    acc2 = _sc_layer2(srcs, dsts, ym1, ycs1, r1)

    out = _stage_c(acc2, w_lin, b_lin)
    return out[:, :OUT]
